# baseline (speedup 1.0000x reference)
import jax
import jax.numpy as jnp
from jax import lax
from jax.experimental import pallas as pl
from jax.experimental.pallas import tpu as pltpu
from jax.experimental.pallas import tpu_sc as plsc

M = 262144
FDIM = 32
HSIZE = 2097152
HMASK = HSIZE - 1
P0, P1, P2 = 73856093, 19349669, 83492791
RES0 = 0.06
RES1 = 0.12

NC = 2
NS = 16
NW = NC * NS
PPW = M // NW
C = 128
NCHUNK = PPW // C
LANES = 16

S0 = 18
NLAT0 = S0 * S0 * S0
PAD0 = 46 * 128
S1 = 10
NLAT1 = S1 * S1 * S1
PAD1 = 8 * 128


def _build_block(feats, h2v, c_out, blk, stride_a, stride_b,
                 hvv, viv, mulv, rows, sem):
    base_l = blk * jnp.int32(128)
    iota = lax.iota(jnp.int32, LANES)
    for i in range(128 // LANES):
        l16 = base_l + jnp.int32(i * LANES) + iota
        lf = l16.astype(jnp.float32)
        ix = (lf / jnp.float32(stride_a)).astype(jnp.int32)
        rem = l16 - ix * jnp.int32(stride_a)
        iy = (rem.astype(jnp.float32) / jnp.float32(stride_b)).astype(
            jnp.int32)
        iz = rem - iy * jnp.int32(stride_b)
        hv = ((ix * jnp.int32(P0) + iy * jnp.int32(P1) + iz * jnp.int32(P2))
              & jnp.int32(HMASK))
        hvv[pl.ds(i * LANES, LANES)] = hv * jnp.int32(2)
    pltpu.async_copy(h2v.at[hvv], viv, sem).wait()
    for i in range(128 // LANES):
        sl = pl.ds(i * LANES, LANES)
        v16 = viv[sl]
        mulv[sl] = jnp.where(v16 >= 0, jnp.float32(1.0), jnp.float32(0.0))
        viv[sl] = jnp.maximum(v16, jnp.int32(0))
    pltpu.async_copy(feats.at[viv], rows, sem).wait()
    for i in range(128 // LANES):
        mvec = mulv[pl.ds(i * LANES, LANES)]
        for j in range(LANES):
            p = i * LANES + j
            m = mvec[j]
            rows[p, 0:16] = m * rows[p, 0:16]
            rows[p, 16:32] = m * rows[p, 16:32]
    pltpu.sync_copy(rows, c_out.at[pl.ds(base_l, 128)])


def _a_body(f0, f1, h0, h1, c0, c1, hvv, viv, mulv, rows, sem):
    cid = lax.axis_index("c")
    sid = lax.axis_index("s")
    wid = sid * jnp.int32(NC) + cid

    _build_block(f0, h0, c0, wid, S0 * S0, S0, hvv, viv, mulv, rows, sem)

    @pl.when(wid < jnp.int32(PAD0 // 128 - NW))
    def _():
        _build_block(f0, h0, c0, wid + jnp.int32(NW), S0 * S0, S0,
                     hvv, viv, mulv, rows, sem)

    @pl.when((wid >= jnp.int32(16)) & (wid < jnp.int32(16 + PAD1 // 128)))
    def _():
        _build_block(f1, h1, c1, wid - jnp.int32(16), S1 * S1, S1,
                     hvv, viv, mulv, rows, sem)


def _b_body(qpf, c0h, c1h, out, qiv, qv, lat0v, lat1v,
            wv0, wv1, rows0, c1v, acc, c0s, sem):
    cid = lax.axis_index("c")
    sid = lax.axis_index("s")
    wid = sid * jnp.int32(NC) + cid
    base0 = wid * jnp.int32(PPW)

    pltpu.sync_copy(c1h, c1v)

    @pl.when(sid == jnp.int32(0))
    def _():
        pltpu.sync_copy(c0h, c0s)

    plsc.subcore_barrier()

    iota = lax.iota(jnp.int32, LANES)

    def _prep(latv, wvv, res, sa, sb):
        for i in range(C // LANES):
            sl = pl.ds(i * LANES, LANES)
            sx = qv[0, sl] / jnp.float32(res)
            sy = qv[1, sl] / jnp.float32(res)
            sz = qv[2, sl] / jnp.float32(res)
            bx = sx.astype(jnp.int32)
            by = sy.astype(jnp.int32)
            bz = sz.astype(jnp.int32)
            fx = sx - bx.astype(jnp.float32)
            fy = sy - by.astype(jnp.float32)
            fz = sz - bz.astype(jnp.float32)
            gx = 1.0 - fx
            gy = 1.0 - fy
            gz = 1.0 - fz
            lbase = bx * jnp.int32(sa) + by * jnp.int32(sb) + bz
            for c in range(8):
                ox, oy, oz = (c >> 2) & 1, (c >> 1) & 1, c & 1
                latv[c, sl] = lbase + jnp.int32(ox * sa + oy * sb + oz)
                wvv[c, sl] = ((fx if ox else gx) * (fy if oy else gy)
                              * (fz if oz else gz))

    def chunk_body(ch, carry):
        base = base0 + ch * jnp.int32(C)
        for i in range(C // LANES):
            sl = pl.ds(i * LANES, LANES)
            t = (base + jnp.int32(i * LANES) + iota) * jnp.int32(3)
            qiv[0, sl] = t
            qiv[1, sl] = t + jnp.int32(1)
            qiv[2, sl] = t + jnp.int32(2)
        qcps = [pltpu.async_copy(qpf.at[qiv.at[d]], qv.at[d], sem)
                for d in range(3)]
        for cp in qcps:
            cp.wait()

        _prep(lat0v, wv0, RES0, S0 * S0, S0)
        cps = [pltpu.async_copy(c0s.at[lat0v.at[c]], rows0.at[c], sem)
               for c in range(8)]

        _prep(lat1v, wv1, RES1, S1 * S1, S1)

        def gbody1(g, carry2):
            p0 = g * jnp.int32(LANES)
            wvecs = [wv1[c, pl.ds(p0, LANES)] for c in range(8)]
            lvecs = [lat1v[c, pl.ds(p0, LANES)] for c in range(8)]
            for j in range(LANES):
                p = p0 + jnp.int32(j)
                r = lvecs[0][j]
                w = wvecs[0][j]
                a_lo = w * c1v[r, 0:16]
                a_hi = w * c1v[r, 16:32]
                for c in range(1, 8):
                    r = lvecs[c][j]
                    w = wvecs[c][j]
                    a_lo = a_lo + w * c1v[r, 0:16]
                    a_hi = a_hi + w * c1v[r, 16:32]
                acc[p, 32:48] = a_lo
                acc[p, 48:64] = a_hi
            return carry2

        lax.fori_loop(jnp.int32(0), jnp.int32(C // LANES), gbody1,
                      jnp.int32(0))

        for cp in cps:
            cp.wait()

        def gbody0(g, carry2):
            p0 = g * jnp.int32(LANES)
            wvecs = [wv0[c, pl.ds(p0, LANES)] for c in range(8)]
            for j in range(LANES):
                p = p0 + jnp.int32(j)
                w = wvecs[0][j]
                a_lo = w * rows0[0, p, 0:16]
                a_hi = w * rows0[0, p, 16:32]
                for c in range(1, 8):
                    w = wvecs[c][j]
                    a_lo = a_lo + w * rows0[c, p, 0:16]
                    a_hi = a_hi + w * rows0[c, p, 16:32]
                acc[p, 0:16] = a_lo
                acc[p, 16:32] = a_hi
            return carry2

        lax.fori_loop(jnp.int32(0), jnp.int32(C // LANES), gbody0,
                      jnp.int32(0))

        pltpu.sync_copy(acc, out.at[pl.ds(base, C)])
        return carry

    lax.fori_loop(jnp.int32(0), jnp.int32(NCHUNK), chunk_body, jnp.int32(0))


@jax.jit
def kernel(query_pts, voxel_features_0, voxel_features_1,
           hash2voxel_0, hash2voxel_1):
    with jax.enable_x64(False):
        h0b = lax.bitcast_convert_type(hash2voxel_0, jnp.int32).reshape(-1)
        h1b = lax.bitcast_convert_type(hash2voxel_1, jnp.int32).reshape(-1)
        qpf = query_pts.reshape(-1)

        mesh = plsc.VectorSubcoreMesh(core_axis_name="c",
                                      subcore_axis_name="s",
                                      num_cores=NC, num_subcores=NS)
        params = pltpu.CompilerParams(use_tc_tiling_on_sc=False)

        build = pl.kernel(
            _a_body,
            out_type=(jax.ShapeDtypeStruct((PAD0, FDIM), jnp.float32),
                      jax.ShapeDtypeStruct((PAD1, FDIM), jnp.float32)),
            mesh=mesh,
            compiler_params=params,
            scratch_types=[
                pltpu.VMEM((128,), jnp.int32),
                pltpu.VMEM((128,), jnp.int32),
                pltpu.VMEM((128,), jnp.float32),
                pltpu.VMEM((128, FDIM), jnp.float32),
                pltpu.SemaphoreType.DMA,
            ],
        )
        c0, c1 = build(voxel_features_0, voxel_features_1, h0b, h1b)

        query = pl.kernel(
            _b_body,
            out_type=jax.ShapeDtypeStruct((M, 2 * FDIM), jnp.float32),
            mesh=mesh,
            compiler_params=params,
            scratch_types=[
                pltpu.VMEM((3, C), jnp.int32),
                pltpu.VMEM((3, C), jnp.float32),
                pltpu.VMEM((8, C), jnp.int32),
                pltpu.VMEM((8, C), jnp.int32),
                pltpu.VMEM((8, C), jnp.float32),
                pltpu.VMEM((8, C), jnp.float32),
                pltpu.VMEM((8, C, FDIM), jnp.float32),
                pltpu.VMEM((PAD1, FDIM), jnp.float32),
                pltpu.VMEM((C, 2 * FDIM), jnp.float32),
                pltpu.VMEM_SHARED((PAD0, FDIM), jnp.float32),
                pltpu.SemaphoreType.DMA,
            ],
        )
        return query(qpf, c0, c1)


# speedup vs baseline: 4.9839x; 4.9839x over previous
import jax
import jax.numpy as jnp
from jax import lax
from jax.experimental import pallas as pl
from jax.experimental.pallas import tpu as pltpu
from jax.experimental.pallas import tpu_sc as plsc

M = 262144
FDIM = 32
HSIZE = 2097152
HMASK = HSIZE - 1
P0, P1, P2 = 73856093, 19349669, 83492791
RES0 = 0.06
RES1 = 0.12

NC = 2
NS = 16
NW = NC * NS
PPW = M // NW
C = 128
NCHUNK = PPW // C
LANES = 16

S0 = 18
NLAT0 = S0 * S0 * S0
PAD0 = 46 * 128
S1 = 10
NLAT1 = S1 * S1 * S1
PAD1 = 8 * 128


def _build_block(feats, h2v, c_out, blk, stride_a, stride_b,
                 hvv, viv, mulv, rows, sem):
    base_l = blk * jnp.int32(128)
    iota = lax.iota(jnp.int32, LANES)
    for i in range(128 // LANES):
        l16 = base_l + jnp.int32(i * LANES) + iota
        lf = l16.astype(jnp.float32)
        ix = (lf / jnp.float32(stride_a)).astype(jnp.int32)
        rem = l16 - ix * jnp.int32(stride_a)
        iy = (rem.astype(jnp.float32) / jnp.float32(stride_b)).astype(
            jnp.int32)
        iz = rem - iy * jnp.int32(stride_b)
        hv = ((ix * jnp.int32(P0) + iy * jnp.int32(P1) + iz * jnp.int32(P2))
              & jnp.int32(HMASK))
        hvv[pl.ds(i * LANES, LANES)] = hv
    pltpu.async_copy(h2v.at[hvv], viv, sem).wait()
    for i in range(128 // LANES):
        sl = pl.ds(i * LANES, LANES)
        v16 = viv[sl]
        mulv[sl] = jnp.where(v16 >= 0, jnp.float32(1.0), jnp.float32(0.0))
        viv[sl] = jnp.maximum(v16, jnp.int32(0))
    pltpu.async_copy(feats.at[viv], rows, sem).wait()
    for i in range(128 // LANES):
        mvec = mulv[pl.ds(i * LANES, LANES)]
        for j in range(LANES):
            p = i * LANES + j
            m = mvec[j]
            rows[p, 0:16] = m * rows[p, 0:16]
            rows[p, 16:32] = m * rows[p, 16:32]
    pltpu.sync_copy(rows, c_out.at[pl.ds(base_l, 128)])


def _a_body(f0, f1, h0, h1, c0, c1, hvv, viv, mulv, rows, sem):
    cid = lax.axis_index("c")
    sid = lax.axis_index("s")
    wid = sid * jnp.int32(NC) + cid

    _build_block(f0, h0, c0, wid, S0 * S0, S0, hvv, viv, mulv, rows, sem)

    @pl.when(wid < jnp.int32(PAD0 // 128 - NW))
    def _():
        _build_block(f0, h0, c0, wid + jnp.int32(NW), S0 * S0, S0,
                     hvv, viv, mulv, rows, sem)

    @pl.when((wid >= jnp.int32(16)) & (wid < jnp.int32(16 + PAD1 // 128)))
    def _():
        _build_block(f1, h1, c1, wid - jnp.int32(16), S1 * S1, S1,
                     hvv, viv, mulv, rows, sem)


def _b_body(qx, qy, qz, c0h, c1h, out, qv0, qv1, qv2, lat0v, lat1v,
            wv0, wv1, rows0, c1v, acc, c0s, sem):
    cid = lax.axis_index("c")
    sid = lax.axis_index("s")
    wid = sid * jnp.int32(NC) + cid
    base0 = wid * jnp.int32(PPW)

    pltpu.sync_copy(c1h, c1v)

    @pl.when(sid == jnp.int32(0))
    def _():
        pltpu.sync_copy(c0h, c0s)

    plsc.subcore_barrier()

    def _prep(latv, wvv, res, sa, sb):
        for i in range(C // LANES):
            sl = pl.ds(i * LANES, LANES)
            sx = qv0[sl] / jnp.float32(res)
            sy = qv1[sl] / jnp.float32(res)
            sz = qv2[sl] / jnp.float32(res)
            bx = sx.astype(jnp.int32)
            by = sy.astype(jnp.int32)
            bz = sz.astype(jnp.int32)
            fx = sx - bx.astype(jnp.float32)
            fy = sy - by.astype(jnp.float32)
            fz = sz - bz.astype(jnp.float32)
            gx = 1.0 - fx
            gy = 1.0 - fy
            gz = 1.0 - fz
            lbase = bx * jnp.int32(sa) + by * jnp.int32(sb) + bz
            for c in range(8):
                ox, oy, oz = (c >> 2) & 1, (c >> 1) & 1, c & 1
                latv[c, sl] = lbase + jnp.int32(ox * sa + oy * sb + oz)
                wvv[c, sl] = ((fx if ox else gx) * (fy if oy else gy)
                              * (fz if oz else gz))

    def chunk_body(ch, carry):
        base = base0 + ch * jnp.int32(C)
        pltpu.sync_copy(qx.at[pl.ds(base, C)], qv0)
        pltpu.sync_copy(qy.at[pl.ds(base, C)], qv1)
        pltpu.sync_copy(qz.at[pl.ds(base, C)], qv2)

        _prep(lat0v, wv0, RES0, S0 * S0, S0)
        cps = [pltpu.async_copy(c0s.at[lat0v.at[c]], rows0.at[c], sem)
               for c in range(8)]

        _prep(lat1v, wv1, RES1, S1 * S1, S1)

        def gbody1(g, carry2):
            p0 = g * jnp.int32(LANES)
            wvecs = [wv1[c, pl.ds(p0, LANES)] for c in range(8)]
            lvecs = [lat1v[c, pl.ds(p0, LANES)] for c in range(8)]
            for j in range(LANES):
                p = p0 + jnp.int32(j)
                r = lvecs[0][j]
                w = wvecs[0][j]
                a_lo = w * c1v[r, 0:16]
                a_hi = w * c1v[r, 16:32]
                for c in range(1, 8):
                    r = lvecs[c][j]
                    w = wvecs[c][j]
                    a_lo = a_lo + w * c1v[r, 0:16]
                    a_hi = a_hi + w * c1v[r, 16:32]
                acc[p, 32:48] = a_lo
                acc[p, 48:64] = a_hi
            return carry2

        lax.fori_loop(jnp.int32(0), jnp.int32(C // LANES), gbody1,
                      jnp.int32(0))

        for cp in cps:
            cp.wait()

        def gbody0(g, carry2):
            p0 = g * jnp.int32(LANES)
            wvecs = [wv0[c, pl.ds(p0, LANES)] for c in range(8)]
            for j in range(LANES):
                p = p0 + jnp.int32(j)
                w = wvecs[0][j]
                a_lo = w * rows0[0, p, 0:16]
                a_hi = w * rows0[0, p, 16:32]
                for c in range(1, 8):
                    w = wvecs[c][j]
                    a_lo = a_lo + w * rows0[c, p, 0:16]
                    a_hi = a_hi + w * rows0[c, p, 16:32]
                acc[p, 0:16] = a_lo
                acc[p, 16:32] = a_hi
            return carry2

        lax.fori_loop(jnp.int32(0), jnp.int32(C // LANES), gbody0,
                      jnp.int32(0))

        pltpu.sync_copy(acc, out.at[pl.ds(base, C)])
        return carry

    lax.fori_loop(jnp.int32(0), jnp.int32(NCHUNK), chunk_body, jnp.int32(0))


@jax.jit
def kernel(query_pts, voxel_features_0, voxel_features_1,
           hash2voxel_0, hash2voxel_1):
    with jax.enable_x64(False):
        qx = query_pts[:, 0]
        qy = query_pts[:, 1]
        qz = query_pts[:, 2]
        h0 = hash2voxel_0.astype(jnp.int32)
        h1 = hash2voxel_1.astype(jnp.int32)

        mesh = plsc.VectorSubcoreMesh(core_axis_name="c",
                                      subcore_axis_name="s",
                                      num_cores=NC, num_subcores=NS)
        params = pltpu.CompilerParams(use_tc_tiling_on_sc=False)

        build = pl.kernel(
            _a_body,
            out_type=(jax.ShapeDtypeStruct((PAD0, FDIM), jnp.float32),
                      jax.ShapeDtypeStruct((PAD1, FDIM), jnp.float32)),
            mesh=mesh,
            compiler_params=params,
            scratch_types=[
                pltpu.VMEM((128,), jnp.int32),
                pltpu.VMEM((128,), jnp.int32),
                pltpu.VMEM((128,), jnp.float32),
                pltpu.VMEM((128, FDIM), jnp.float32),
                pltpu.SemaphoreType.DMA,
            ],
        )
        c0, c1 = build(voxel_features_0, voxel_features_1, h0, h1)

        query = pl.kernel(
            _b_body,
            out_type=jax.ShapeDtypeStruct((M, 2 * FDIM), jnp.float32),
            mesh=mesh,
            compiler_params=params,
            scratch_types=[
                pltpu.VMEM((C,), jnp.float32),
                pltpu.VMEM((C,), jnp.float32),
                pltpu.VMEM((C,), jnp.float32),
                pltpu.VMEM((8, C), jnp.int32),
                pltpu.VMEM((8, C), jnp.int32),
                pltpu.VMEM((8, C), jnp.float32),
                pltpu.VMEM((8, C), jnp.float32),
                pltpu.VMEM((8, C, FDIM), jnp.float32),
                pltpu.VMEM((PAD1, FDIM), jnp.float32),
                pltpu.VMEM((C, 2 * FDIM), jnp.float32),
                pltpu.VMEM_SHARED((PAD0, FDIM), jnp.float32),
                pltpu.SemaphoreType.DMA,
            ],
        )
        return query(qx, qy, qz, c0, c1)
